# baseline (speedup 1.0000x reference)
import functools

import jax
import jax.numpy as jnp
from jax import lax
from jax.experimental import pallas as pl
from jax.experimental.pallas import tpu as pltpu
from jax.experimental.pallas import tpu_sc as plsc

NC, NS = 2, 16
NW = NC * NS


def sc_add(x, emb_table):
    B, T, D = x.shape
    per_w = B // NW
    half = per_w // 2
    mesh = plsc.VectorSubcoreMesh(core_axis_name="c", subcore_axis_name="s")

    @functools.partial(
        pl.kernel,
        mesh=mesh,
        out_type=jax.ShapeDtypeStruct((B, T, D), jnp.float32),
        scratch_types=[
            pltpu.VMEM((T, D), jnp.float32),
            pltpu.VMEM((T, D), jnp.float32),
            pltpu.VMEM((T, D), jnp.float32),
            pltpu.SemaphoreType.DMA,
            pltpu.SemaphoreType.DMA,
            pltpu.SemaphoreType.DMA,
            pltpu.SemaphoreType.DMA,
        ],
    )
    def k(x_hbm, emb_hbm, out_hbm, emb_v, buf0, buf1, si0, si1, so0, so1):
        wid = lax.axis_index("s") * NC + lax.axis_index("c")
        base = wid * per_w
        pltpu.sync_copy(emb_hbm, emb_v)

        def add_table(buf):
            @plsc.parallel_loop(0, T, unroll=2)
            def _(r):
                for j in range(D // 16):
                    sl = pl.ds(j * 16, 16)
                    buf[r, sl] = buf[r, sl] + emb_v[r, sl]

        pltpu.async_copy(x_hbm.at[base], buf0, si0)
        pltpu.async_copy(x_hbm.at[base + 1], buf1, si1)

        def body(i, carry):
            b0 = base + 2 * i
            b1 = b0 + 1
            pltpu.make_async_copy(x_hbm.at[b0], buf0, si0).wait()
            add_table(buf0)
            pltpu.async_copy(buf0, out_hbm.at[b0], so0)
            pltpu.make_async_copy(x_hbm.at[b1], buf1, si1).wait()
            add_table(buf1)
            pltpu.async_copy(buf1, out_hbm.at[b1], so1)

            @pl.when(i < half - 1)
            def _():
                pltpu.make_async_copy(buf0, out_hbm.at[b0], so0).wait()
                pltpu.async_copy(x_hbm.at[b0 + 2], buf0, si0)
                pltpu.make_async_copy(buf1, out_hbm.at[b1], so1).wait()
                pltpu.async_copy(x_hbm.at[b1 + 2], buf1, si1)

            return carry

        lax.fori_loop(0, half, body, 0)
        last0 = base + per_w - 2
        last1 = base + per_w - 1
        pltpu.make_async_copy(buf0, out_hbm.at[last0], so0).wait()
        pltpu.make_async_copy(buf1, out_hbm.at[last1], so1).wait()

    return k(x, emb_table[:T])


def kernel(x, emb_table):
    return sc_add(x, emb_table)


# speedup vs baseline: 1.4296x; 1.4296x over previous
import functools

import jax
import jax.numpy as jnp
from jax import lax
from jax.experimental import pallas as pl
from jax.experimental.pallas import tpu as pltpu
from jax.experimental.pallas import tpu_sc as plsc

_NC, _NS = 2, 16
_NW = _NC * _NS


def _sc_gather(emb_table, positions):
    T = positions.shape[0]
    D = emb_table.shape[1]
    rows_per_w = 8
    active = T // rows_per_w
    mesh = plsc.VectorSubcoreMesh(core_axis_name="c", subcore_axis_name="s")

    @functools.partial(
        pl.kernel,
        mesh=mesh,
        out_type=jax.ShapeDtypeStruct((T, D), jnp.float32),
        scratch_types=[
            pltpu.VMEM((rows_per_w,), jnp.int32),
            pltpu.VMEM((rows_per_w, D), jnp.float32),
            pltpu.SemaphoreType.DMA,
        ],
    )
    def k(emb_hbm, pos_hbm, out_hbm, idx_v, rows_v, sem):
        wid = lax.axis_index("s") * _NC + lax.axis_index("c")

        @pl.when(wid < active)
        def _():
            base = wid * rows_per_w
            pltpu.sync_copy(pos_hbm.at[pl.ds(base, rows_per_w)], idx_v)
            pltpu.async_copy(emb_hbm.at[idx_v], rows_v, sem).wait()
            pltpu.sync_copy(rows_v, out_hbm.at[pl.ds(base, rows_per_w)])

    return k(emb_table, positions)


def _add_body(x_ref, emb_ref, o_ref):
    o_ref[...] = x_ref[...] + emb_ref[...][None, :, :]


def _tc_add(x, pos_emb):
    B, T, D = x.shape
    B_BLK = 128
    return pl.pallas_call(
        _add_body,
        grid=(B // B_BLK,),
        in_specs=[
            pl.BlockSpec((B_BLK, T, D), lambda i: (i, 0, 0)),
            pl.BlockSpec((T, D), lambda i: (0, 0)),
        ],
        out_specs=pl.BlockSpec((B_BLK, T, D), lambda i: (i, 0, 0)),
        out_shape=jax.ShapeDtypeStruct((B, T, D), x.dtype),
    )(x, pos_emb)


def kernel(x, emb_table):
    T = x.shape[1]
    positions = jnp.arange(T, dtype=jnp.int32)
    pos_emb = _sc_gather(emb_table, positions)
    return _tc_add(x, pos_emb)


# speedup vs baseline: 1.4370x; 1.0052x over previous
import functools

import jax
import jax.numpy as jnp
from jax import lax
from jax.experimental import pallas as pl
from jax.experimental.pallas import tpu as pltpu
from jax.experimental.pallas import tpu_sc as plsc

_NC, _NS = 2, 16
_NW = _NC * _NS


def _sc_gather(emb_table, positions):
    T = positions.shape[0]
    D = emb_table.shape[1]
    full_rows, tail_rows = 16, 8
    n_full = T // full_rows
    tail_base = n_full * full_rows
    mesh = plsc.VectorSubcoreMesh(
        core_axis_name="c", subcore_axis_name="s", num_cores=1
    )

    @functools.partial(
        pl.kernel,
        mesh=mesh,
        out_type=jax.ShapeDtypeStruct((T, D), jnp.float32),
        scratch_types=[
            pltpu.VMEM((full_rows,), jnp.int32),
            pltpu.VMEM((full_rows, D), jnp.float32),
            pltpu.SemaphoreType.DMA,
        ],
    )
    def k(emb_hbm, pos_hbm, out_hbm, idx_v, rows_v, sem):
        wid = lax.axis_index("s")

        @pl.when(wid < n_full)
        def _():
            base = wid * full_rows
            pltpu.sync_copy(pos_hbm.at[pl.ds(base, full_rows)], idx_v)
            pltpu.async_copy(emb_hbm.at[idx_v], rows_v, sem).wait()
            pltpu.sync_copy(rows_v, out_hbm.at[pl.ds(base, full_rows)])

        @pl.when(wid == n_full)
        def _():
            idx8 = idx_v.at[pl.ds(0, tail_rows)]
            rows8 = rows_v.at[pl.ds(0, tail_rows)]
            pltpu.sync_copy(pos_hbm.at[pl.ds(tail_base, tail_rows)], idx8)
            pltpu.async_copy(emb_hbm.at[idx8], rows8, sem).wait()
            pltpu.sync_copy(rows8, out_hbm.at[pl.ds(tail_base, tail_rows)])

    return k(emb_table, positions)


def _add_body(x_ref, emb_ref, o_ref):
    o_ref[...] = x_ref[...] + emb_ref[...][None, :, :]


def _tc_add(x, pos_emb):
    B, T, D = x.shape
    B_BLK = 128
    return pl.pallas_call(
        _add_body,
        grid=(B // B_BLK,),
        in_specs=[
            pl.BlockSpec((B_BLK, T, D), lambda i: (i, 0, 0)),
            pl.BlockSpec((T, D), lambda i: (0, 0)),
        ],
        out_specs=pl.BlockSpec((B_BLK, T, D), lambda i: (i, 0, 0)),
        out_shape=jax.ShapeDtypeStruct((B, T, D), x.dtype),
    )(x, pos_emb)


def kernel(x, emb_table):
    T = x.shape[1]
    positions = jnp.arange(T, dtype=jnp.int32)
    pos_emb = _sc_gather(emb_table, positions)
    return _tc_add(x, pos_emb)


# speedup vs baseline: 1.4397x; 1.0019x over previous
import functools

import jax
import jax.numpy as jnp
from jax import lax
from jax.experimental import pallas as pl
from jax.experimental.pallas import tpu as pltpu
from jax.experimental.pallas import tpu_sc as plsc

_NC, _NS = 2, 16
_NW = _NC * _NS


def _sc_gather(emb_table, T):
    D = emb_table.shape[1]
    rows = 16
    n_w = (T + rows - 1) // rows
    mesh = plsc.VectorSubcoreMesh(
        core_axis_name="c", subcore_axis_name="s", num_cores=1
    )

    @functools.partial(
        pl.kernel,
        mesh=mesh,
        out_type=jax.ShapeDtypeStruct((T, D), jnp.float32),
        scratch_types=[
            pltpu.VMEM((rows, D), jnp.float32),
            pltpu.SemaphoreType.DMA,
        ],
    )
    def k(emb_hbm, out_hbm, rows_v, sem):
        wid = lax.axis_index("s")

        @pl.when(wid < n_w)
        def _():
            base = jnp.minimum(wid * rows, T - rows)
            idx = lax.iota(jnp.int32, rows) + base
            pltpu.async_copy(emb_hbm.at[idx], rows_v, sem).wait()
            pltpu.sync_copy(rows_v, out_hbm.at[pl.ds(base, rows)])

    return k(emb_table)


def _add_body(x_ref, emb_ref, o_ref):
    o_ref[...] = x_ref[...] + emb_ref[...][None, :, :]


def _tc_add(x, pos_emb):
    B, T, D = x.shape
    B_BLK = 128
    return pl.pallas_call(
        _add_body,
        grid=(B // B_BLK,),
        in_specs=[
            pl.BlockSpec((B_BLK, T, D), lambda i: (i, 0, 0)),
            pl.BlockSpec((T, D), lambda i: (0, 0)),
        ],
        out_specs=pl.BlockSpec((B_BLK, T, D), lambda i: (i, 0, 0)),
        out_shape=jax.ShapeDtypeStruct((B, T, D), x.dtype),
    )(x, pos_emb)


def kernel(x, emb_table):
    T = x.shape[1]
    pos_emb = _sc_gather(emb_table, T)
    return _tc_add(x, pos_emb)
